# Initial kernel scaffold; baseline (speedup 1.0000x reference)
import jax
import jax.numpy as jnp
from jax.experimental import pallas as pl


def kernel(feat, edge_index, W0_0, b0_0, W0_1, b0_1, ln_g0, ln_b0, W1_0, b1_0, W1_1, b1_1, ln_g1, ln_b1, ws1, ws2):
    raise NotImplementedError("write your pallas kernel here")


# speedup vs baseline: 1.8879x; 1.8879x over previous
import jax
import jax.numpy as jnp
from jax import lax
from jax.experimental import pallas as pl
from jax.experimental.pallas import tpu as pltpu
from jax.experimental.pallas import tpu_sc as plsc

N = 10000
E = 160000
R = 2
D = 256
DA = 16
Q = 64
NSUB = 16
NCORE = 2
EPT = E // NSUB
CHUNK = 80
NCHUNK = EPT // CHUNK
NPAD = 10240
ROWS_PT = NPAD // NSUB
ZROWS = 128


def _sc_aggregate_body(featq, src0, dst0, src1, dst1, agg_out, cnt_out,
                       sidx, didx, rows_v, ones_v, zrow_v, zcnt_v,
                       agg_sh, cnt_sh, sem):
    cid = lax.axis_index("c")
    sid = lax.axis_index("s")

    def init_ones(i, _):
        ones_v[i] = jnp.ones((16,), jnp.float32)
        return ()
    lax.fori_loop(0, CHUNK, init_ones, ())

    def init_zrow(i, _):
        r = i // (Q // 16)
        j = i % (Q // 16)
        zrow_v[r, pl.ds(j * 16, 16)] = jnp.zeros((16,), jnp.float32)
        return ()
    lax.fori_loop(0, ZROWS * (Q // 16), init_zrow, ())

    def init_zcnt(i, _):
        zcnt_v[i] = jnp.zeros((16,), jnp.float32)
        return ()
    lax.fori_loop(0, ROWS_PT, init_zcnt, ())

    row0 = sid * ROWS_PT
    for r, (src, dst) in enumerate(((src0, dst0), (src1, dst1))):
        for p in range(2):
            for k in range(ROWS_PT // ZROWS):
                pltpu.sync_copy(zrow_v, agg_sh.at[pl.ds(row0 + k * ZROWS, ZROWS)])
            if p == 0:
                pltpu.sync_copy(zcnt_v, cnt_sh.at[pl.ds(row0, ROWS_PT)])
            plsc.subcore_barrier()

            def chunk_body(i, _):
                base = sid * EPT + i * CHUNK
                pltpu.sync_copy(src.at[pl.ds(base, CHUNK)], sidx)
                pltpu.sync_copy(dst.at[pl.ds(base, CHUNK)], didx.at[0])
                for j in range(CHUNK // 16):
                    s = sidx[pl.ds(j * 16, 16)]
                    sidx[pl.ds(j * 16, 16)] = s * 4 + (2 * cid + p)
                pltpu.async_copy(featq.at[sidx], rows_v, sem).wait()
                pltpu.sync_copy(rows_v, agg_sh.at[didx.at[0]], add=True)
                if p == 0:
                    pltpu.sync_copy(ones_v, cnt_sh.at[didx.at[0]], add=True)
                return ()
            lax.fori_loop(0, NCHUNK, chunk_body, ())
            plsc.subcore_barrier()

            pltpu.sync_copy(agg_sh.at[pl.ds(row0, ROWS_PT)],
                            agg_out.at[r, 2 * cid + p, pl.ds(row0, ROWS_PT)])
            if p == 0:
                @pl.when(cid == 0)
                def _():
                    pltpu.sync_copy(cnt_sh.at[pl.ds(row0, ROWS_PT)],
                                    cnt_out.at[r, pl.ds(row0, ROWS_PT)])
            plsc.subcore_barrier()


def _make_sc_aggregate():
    mesh = plsc.VectorSubcoreMesh(core_axis_name="c", subcore_axis_name="s")
    return pl.kernel(
        _sc_aggregate_body,
        out_type=(
            jax.ShapeDtypeStruct((R, 4, NPAD, Q), jnp.float32),
            jax.ShapeDtypeStruct((R, NPAD, DA), jnp.float32),
        ),
        mesh=mesh,
        scratch_types=[
            pltpu.VMEM((CHUNK,), jnp.int32),
            pltpu.VMEM((1, CHUNK), jnp.int32),
            pltpu.VMEM((CHUNK, Q), jnp.float32),
            pltpu.VMEM((CHUNK, DA), jnp.float32),
            pltpu.VMEM((ZROWS, Q), jnp.float32),
            pltpu.VMEM((ROWS_PT, DA), jnp.float32),
            pltpu.VMEM_SHARED((NPAD, Q), jnp.float32),
            pltpu.VMEM_SHARED((NPAD, DA), jnp.float32),
            pltpu.SemaphoreType.DMA,
        ],
        compiler_params=pltpu.CompilerParams(use_tc_tiling_on_sc=False),
    )


BLK = 1000


def _tc_dense_body(feat_ref, agg_ref, cnt_ref,
                   W00, b00, W01, b01, g0, lb0,
                   W10, b10, W11, b11, g1, lb1,
                   ws1_ref, ws2_ref, out_ref):
    i = pl.program_id(0)
    feat = feat_ref[...]
    params = ((W00, b00, W01, b01, g0, lb0),
              (W10, b10, W11, b11, g1, lb1))

    def layer_norm(x, g, b):
        mu = jnp.mean(x, axis=-1, keepdims=True)
        var = jnp.mean((x - mu) ** 2, axis=-1, keepdims=True)
        return (x - mu) / jnp.sqrt(var + 1e-5) * g + b

    hs = []
    ss = []
    for r in range(R):
        inv = 1.0 / jnp.maximum(cnt_ref[r][:, 0:1], 1.0)
        Wa, ba, Wb, bb, g, b = params[r]
        Wa = Wa[...]
        ga = g[...]
        bl = b[...]
        x = jnp.dot(feat, Wa, preferred_element_type=jnp.float32)
        for q in range(4):
            x += jnp.dot(agg_ref[r, q] * inv, Wa[q * Q:(q + 1) * Q],
                         preferred_element_type=jnp.float32)
        x = jax.nn.relu(layer_norm(x + ba[...], ga, bl))
        x = jnp.dot(x, Wb[...], preferred_element_type=jnp.float32) + bb[...]
        x = jax.nn.relu(layer_norm(x, ga, bl))
        hs.append(x)
        t = jnp.tanh(jnp.dot(x, ws1_ref[r], preferred_element_type=jnp.float32))
        s = jnp.dot(t, ws2_ref[r][:, None],
                    preferred_element_type=jnp.float32)
        ss.append(s)

    m = jnp.maximum(ss[0], ss[1])
    e0 = jnp.exp(ss[0] - m)
    e1 = jnp.exp(ss[1] - m)
    tot = e0 + e1
    h_out = (e0 / tot) * hs[0] + (e1 / tot) * hs[1]
    blk = jnp.sum(h_out, axis=0, keepdims=True) * (1.0 / N)

    @pl.when(i == 0)
    def _():
        out_ref[...] = jnp.zeros_like(out_ref)
    out_ref[...] += blk


def _make_tc_dense():
    full = lambda *shape: pl.BlockSpec(shape, lambda i: (0,) * len(shape))
    row_blk = pl.BlockSpec((BLK, D), lambda i: (i, 0))
    w_spec = full(D, D)
    b_spec = full(D)
    return pl.pallas_call(
        _tc_dense_body,
        grid=(N // BLK,),
        in_specs=[
            row_blk,
            pl.BlockSpec((R, 4, BLK, Q), lambda i: (0, 0, i, 0)),
            pl.BlockSpec((R, BLK, DA), lambda i: (0, i, 0)),
            w_spec, b_spec, w_spec, b_spec, b_spec, b_spec,
            w_spec, b_spec, w_spec, b_spec, b_spec, b_spec,
            full(R, D, DA),
            full(R, DA),
        ],
        out_specs=pl.BlockSpec((1, D), lambda i: (0, 0)),
        out_shape=jax.ShapeDtypeStruct((1, D), jnp.float32),
    )


@jax.jit
def kernel(feat, edge_index, W0_0, b0_0, W0_1, b0_1, ln_g0, ln_b0,
           W1_0, b1_0, W1_1, b1_1, ln_g1, ln_b1, ws1, ws2):
    edge_index = edge_index.astype(jnp.int32)
    featq = feat.reshape(4 * N, Q)
    agg, cnt = _make_sc_aggregate()(
        featq, edge_index[0, 0], edge_index[0, 1],
        edge_index[1, 0], edge_index[1, 1])
    out = _make_tc_dense()(
        feat, agg, cnt,
        W0_0, b0_0, W0_1, b0_1, ln_g0, ln_b0,
        W1_0, b1_0, W1_1, b1_1, ln_g1, ln_b1,
        ws1, ws2.reshape(R, DA))
    return out


# speedup vs baseline: 4.9644x; 2.6296x over previous
import jax
import jax.numpy as jnp
from jax import lax
from jax.experimental import pallas as pl
from jax.experimental.pallas import tpu as pltpu
from jax.experimental.pallas import tpu_sc as plsc

N = 10000
E = 160000
R = 2
D = 256
DA = 16
Q = 64
NSUB = 16
EPT = E // NSUB
CHUNK = 125
NCHUNK = EPT // CHUNK
NB = 4
NPAD = 10240
ROWS_PT = NPAD // NSUB
ZROWS = 128


def _sc_aggregate_body(t0, t1, src0, dst0, src1, dst1, agg_out, cnt_out,
                       sidx, didx, rows, ones_v, zrow_v, zcnt_v,
                       agg_sh, cnt_sh, gsem, ssem):
    cid = lax.axis_index("c")
    sid = lax.axis_index("s")
    coff = cid * N
    row0 = sid * ROWS_PT

    def init_ones(i, _):
        ones_v[i] = jnp.ones((16,), jnp.float32)
        return ()
    lax.fori_loop(0, CHUNK, init_ones, ())

    def init_zrow(i, _):
        r = i // (Q // 16)
        j = i % (Q // 16)
        zrow_v[r, pl.ds(j * 16, 16)] = jnp.zeros((16,), jnp.float32)
        return ()
    lax.fori_loop(0, ZROWS * (Q // 16), init_zrow, ())

    def init_zcnt(i, _):
        zcnt_v[i] = jnp.zeros((16,), jnp.float32)
        return ()
    lax.fori_loop(0, ROWS_PT, init_zcnt, ())

    for r, (src, dst) in enumerate(((src0, dst0), (src1, dst1))):
        pltpu.sync_copy(src.at[sid], sidx)
        pltpu.sync_copy(dst.at[sid], didx)

        def shift_row(i, _):
            for k in range(CHUNK // 16):
                s = sidx[i, pl.ds(k * 16, 16)]
                sidx[i, pl.ds(k * 16, 16)] = s + coff
            s = sidx[i, pl.ds(CHUNK - 16, 16)]
            sidx[i, pl.ds(CHUNK - 16, 16)] = s + coff
            return ()
        if CHUNK % 16 == 0:
            lax.fori_loop(0, NCHUNK, shift_row, ())
        else:
            def shift_row2(i, _):
                nfull = CHUNK // 16
                for k in range(nfull):
                    s = sidx[i, pl.ds(k * 16, 16)]
                    sidx[i, pl.ds(k * 16, 16)] = s + coff
                tail = CHUNK - nfull * 16
                s = sidx[i, pl.ds(nfull * 16 - (16 - tail), 16)]
                mask = lax.iota(jnp.int32, 16) >= (16 - tail)
                sidx[i, pl.ds(nfull * 16 - (16 - tail), 16)] = jnp.where(
                    mask, s + coff, s)
                return ()
            lax.fori_loop(0, NCHUNK, shift_row2, ())

        for p in range(2):
            table = (t0, t1)[p]
            for k in range(ROWS_PT // ZROWS):
                pltpu.sync_copy(zrow_v, agg_sh.at[pl.ds(row0 + k * ZROWS, ZROWS)])
            if p == 0:
                pltpu.sync_copy(zcnt_v, cnt_sh.at[pl.ds(row0, ROWS_PT)])
            plsc.subcore_barrier()

            def outer(j, _):
                gds = []
                for b in range(NB):
                    i = j * NB + b

                    @pl.when(j > 0)
                    def _(b=b):
                        pltpu.make_async_copy(
                            table.at[pl.ds(0, CHUNK)], rows.at[b],
                            ssem.at[b]).wait()
                    gds.append(pltpu.async_copy(
                        table.at[sidx.at[i]], rows.at[b], gsem.at[b]))
                for b in range(NB):
                    i = j * NB + b
                    gds[b].wait()
                    pltpu.async_copy(rows.at[b], agg_sh.at[didx.at[i]],
                                     ssem.at[b], add=True)
                    if p == 0:
                        pltpu.sync_copy(ones_v, cnt_sh.at[didx.at[i]],
                                        add=True)
                return ()
            lax.fori_loop(0, NCHUNK // NB, outer, ())
            for b in range(NB):
                pltpu.make_async_copy(table.at[pl.ds(0, CHUNK)], rows.at[b],
                                      ssem.at[b]).wait()
            plsc.subcore_barrier()

            pltpu.sync_copy(agg_sh.at[pl.ds(row0, ROWS_PT)],
                            agg_out.at[r, 2 * cid + p, pl.ds(row0, ROWS_PT)])
            if p == 0:
                @pl.when(cid == 0)
                def _():
                    pltpu.sync_copy(cnt_sh.at[pl.ds(row0, ROWS_PT)],
                                    cnt_out.at[r, pl.ds(row0, ROWS_PT)])
            plsc.subcore_barrier()


def _make_sc_aggregate():
    mesh = plsc.VectorSubcoreMesh(core_axis_name="c", subcore_axis_name="s")
    return pl.kernel(
        _sc_aggregate_body,
        out_type=(
            jax.ShapeDtypeStruct((R, 4, NPAD, Q), jnp.float32),
            jax.ShapeDtypeStruct((R, NPAD, DA), jnp.float32),
        ),
        mesh=mesh,
        scratch_types=[
            pltpu.VMEM((NCHUNK, CHUNK), jnp.int32),
            pltpu.VMEM((NCHUNK, CHUNK), jnp.int32),
            pltpu.VMEM((NB, CHUNK, Q), jnp.float32),
            pltpu.VMEM((CHUNK, DA), jnp.float32),
            pltpu.VMEM((ZROWS, Q), jnp.float32),
            pltpu.VMEM((ROWS_PT, DA), jnp.float32),
            pltpu.VMEM_SHARED((NPAD, Q), jnp.float32),
            pltpu.VMEM_SHARED((NPAD, DA), jnp.float32),
            pltpu.SemaphoreType.DMA((NB,)),
            pltpu.SemaphoreType.DMA((NB,)),
        ],
        compiler_params=pltpu.CompilerParams(use_tc_tiling_on_sc=False),
    )


BLK = 1000


def _tc_dense_body(feat_ref, agg_ref, cnt_ref,
                   W00, b00, W01, b01, g0, lb0,
                   W10, b10, W11, b11, g1, lb1,
                   ws1_ref, ws2_ref, out_ref):
    i = pl.program_id(0)
    feat = feat_ref[...]
    params = ((W00, b00, W01, b01, g0, lb0),
              (W10, b10, W11, b11, g1, lb1))

    def layer_norm(x, g, b):
        mu = jnp.mean(x, axis=-1, keepdims=True)
        var = jnp.mean((x - mu) ** 2, axis=-1, keepdims=True)
        return (x - mu) / jnp.sqrt(var + 1e-5) * g + b

    hs = []
    ss = []
    for r in range(R):
        inv = 1.0 / jnp.maximum(cnt_ref[r][:, 0:1], 1.0)
        Wa, ba, Wb, bb, g, b = params[r]
        Wa = Wa[...]
        ga = g[...]
        bl = b[...]
        x = jnp.dot(feat, Wa, preferred_element_type=jnp.float32)
        for q in range(4):
            x += jnp.dot(agg_ref[r, q] * inv, Wa[q * Q:(q + 1) * Q],
                         preferred_element_type=jnp.float32)
        x = jax.nn.relu(layer_norm(x + ba[...], ga, bl))
        x = jnp.dot(x, Wb[...], preferred_element_type=jnp.float32) + bb[...]
        x = jax.nn.relu(layer_norm(x, ga, bl))
        hs.append(x)
        t = jnp.tanh(jnp.dot(x, ws1_ref[r], preferred_element_type=jnp.float32))
        s = jnp.dot(t, ws2_ref[r][:, None],
                    preferred_element_type=jnp.float32)
        ss.append(s)

    m = jnp.maximum(ss[0], ss[1])
    e0 = jnp.exp(ss[0] - m)
    e1 = jnp.exp(ss[1] - m)
    tot = e0 + e1
    h_out = (e0 / tot) * hs[0] + (e1 / tot) * hs[1]
    blk = jnp.sum(h_out, axis=0, keepdims=True) * (1.0 / N)

    @pl.when(i == 0)
    def _():
        out_ref[...] = jnp.zeros_like(out_ref)
    out_ref[...] += blk


def _make_tc_dense():
    full = lambda *shape: pl.BlockSpec(shape, lambda i: (0,) * len(shape))
    row_blk = pl.BlockSpec((BLK, D), lambda i: (i, 0))
    w_spec = full(D, D)
    b_spec = full(D)
    return pl.pallas_call(
        _tc_dense_body,
        grid=(N // BLK,),
        in_specs=[
            row_blk,
            pl.BlockSpec((R, 4, BLK, Q), lambda i: (0, 0, i, 0)),
            pl.BlockSpec((R, BLK, DA), lambda i: (0, i, 0)),
            w_spec, b_spec, w_spec, b_spec, b_spec, b_spec,
            w_spec, b_spec, w_spec, b_spec, b_spec, b_spec,
            full(R, D, DA),
            full(R, DA),
        ],
        out_specs=pl.BlockSpec((1, D), lambda i: (0, 0)),
        out_shape=jax.ShapeDtypeStruct((1, D), jnp.float32),
    )


@jax.jit
def kernel(feat, edge_index, W0_0, b0_0, W0_1, b0_1, ln_g0, ln_b0,
           W1_0, b1_0, W1_1, b1_1, ln_g1, ln_b1, ws1, ws2):
    edge_index = edge_index.astype(jnp.int32)
    t0 = jnp.concatenate([feat[:, 0:Q], feat[:, 2 * Q:3 * Q]], axis=0)
    t1 = jnp.concatenate([feat[:, Q:2 * Q], feat[:, 3 * Q:4 * Q]], axis=0)
    eidx = edge_index.reshape(R, 2, NSUB, NCHUNK, CHUNK)
    agg, cnt = _make_sc_aggregate()(
        t0, t1, eidx[0, 0], eidx[0, 1], eidx[1, 0], eidx[1, 1])
    out = _make_tc_dense()(
        feat, agg, cnt,
        W0_0, b0_0, W0_1, b0_1, ln_g0, ln_b0,
        W1_0, b1_0, W1_1, b1_1, ln_g1, ln_b1,
        ws1, ws2.reshape(R, DA))
    return out
